# baseline (speedup 1.0000x reference)
import jax
import jax.numpy as jnp
from jax.experimental import pallas as pl
from jax.experimental.pallas import tpu as pltpu

SEQ = 1024
KC = 128
VOC = 2 * KC + 1
PADL = SEQ - KC
DIM = 64
BLOCK = 32


def _body(tk_ref, tv_ref, ok_ref, ov_ref, fk8, fv8, sem_k, sem_v):
    pid = pl.program_id(0)

    @pl.when(pid == 0)
    def _build():
        for t_ref, f8 in ((tk_ref, fk8), (tv_ref, fv8)):
            for p in range(8):
                f8[p, 0:PADL - p, :] = jnp.broadcast_to(
                    t_ref[0:1, :], (PADL - p, DIM))
                f8[p, PADL - p:PADL - p + VOC, :] = t_ref[...]
                f8[p, PADL - p + VOC:2 * SEQ, :] = jnp.broadcast_to(
                    t_ref[VOC - 1:VOC, :], (2 * SEQ - PADL + p - VOC, DIM))

    for r in range(BLOCK):
        i = pid * BLOCK + r
        p = (8 - r % 8) % 8
        base = SEQ - pid * BLOCK - r - p
        pltpu.make_async_copy(
            fk8.at[p, pl.ds(base, SEQ), :], ok_ref.at[i], sem_k).start()
        pltpu.make_async_copy(
            fv8.at[p, pl.ds(base, SEQ), :], ov_ref.at[i], sem_v).start()
    for r in range(BLOCK):
        i = pid * BLOCK + r
        p = (8 - r % 8) % 8
        base = SEQ - pid * BLOCK - r - p
        pltpu.make_async_copy(
            fk8.at[p, pl.ds(base, SEQ), :], ok_ref.at[i], sem_k).wait()
        pltpu.make_async_copy(
            fv8.at[p, pl.ds(base, SEQ), :], ov_ref.at[i], sem_v).wait()


def kernel(seq_len, table_k, table_v):
    del seq_len
    out = pl.pallas_call(
        _body,
        grid=(SEQ // BLOCK,),
        in_specs=[
            pl.BlockSpec((VOC, DIM), lambda b: (0, 0)),
            pl.BlockSpec((VOC, DIM), lambda b: (0, 0)),
        ],
        out_specs=[
            pl.BlockSpec(memory_space=pl.ANY),
            pl.BlockSpec(memory_space=pl.ANY),
        ],
        out_shape=[
            jax.ShapeDtypeStruct((SEQ, SEQ, DIM), jnp.float32),
            jax.ShapeDtypeStruct((SEQ, SEQ, DIM), jnp.float32),
        ],
        scratch_shapes=[
            pltpu.VMEM((8, 2 * SEQ, DIM), jnp.float32),
            pltpu.VMEM((8, 2 * SEQ, DIM), jnp.float32),
            pltpu.SemaphoreType.DMA,
            pltpu.SemaphoreType.DMA,
        ],
    )(table_k, table_v)
    return (out[0], out[1])
